# Initial kernel scaffold; baseline (speedup 1.0000x reference)
import jax
import jax.numpy as jnp
from jax.experimental import pallas as pl


def kernel(class_logits, box_regression, proposals):
    raise NotImplementedError("write your pallas kernel here")


# speedup vs baseline: 565.7476x; 565.7476x over previous
import functools
import math

import jax
import jax.numpy as jnp
from jax import lax
from jax.experimental import pallas as pl
from jax.experimental.pallas import tpu as pltpu
from jax.experimental.pallas import tpu_sc as plsc

N = 5000
NP = 5120
NC = 81
NCLS = 80
SCORE_THRESH = 0.05
NMS_THRESH = 0.5
DETS = 100
CLIP = float(math.log(1000.0 / 16.0))
IMG_W, IMG_H = 1333.0, 800.0
NEG = -1.0e30
CAP = 5136


def _score_decode_body(logits_ref, dx_ref, dy_ref, dw_ref, dh_ref, props_ref,
                       sc_ref, x1_ref, y1_ref, x2_ref, y2_ref):
    l = logits_ref[...]
    m = jnp.max(l, axis=0, keepdims=True)
    e = jnp.exp(l - m)
    s = jnp.sum(e, axis=0, keepdims=True)
    prob = e / s
    col = lax.broadcasted_iota(jnp.int32, (1, NP), 1)
    sc_ref[...] = jnp.where(col < N, prob[1:NC], 0.0)

    px1 = props_ref[0:1, :]
    py1 = props_ref[1:2, :]
    px2 = props_ref[2:3, :]
    py2 = props_ref[3:4, :]
    w = px2 - px1 + 1.0
    h = py2 - py1 + 1.0
    cx = px1 + 0.5 * w
    cy = py1 + 0.5 * h
    dx = dx_ref[...] / 10.0
    dy = dy_ref[...] / 10.0
    dw = jnp.minimum(dw_ref[...] / 5.0, CLIP)
    dh = jnp.minimum(dh_ref[...] / 5.0, CLIP)
    pcx = dx * w + cx
    pcy = dy * h + cy
    pw = jnp.exp(dw) * w
    ph = jnp.exp(dh) * h
    x1_ref[...] = jnp.clip(pcx - 0.5 * pw, 0.0, IMG_W - 1.0)
    y1_ref[...] = jnp.clip(pcy - 0.5 * ph, 0.0, IMG_H - 1.0)
    x2_ref[...] = jnp.clip(pcx + 0.5 * pw - 1.0, 0.0, IMG_W - 1.0)
    y2_ref[...] = jnp.clip(pcy + 0.5 * ph - 1.0, 0.0, IMG_H - 1.0)


def _score_decode(logits_t, dx_t, dy_t, dw_t, dh_t, props_t):
    out = jax.ShapeDtypeStruct((NCLS, NP), jnp.float32)
    return pl.pallas_call(
        _score_decode_body,
        out_shape=[out] * 5,
    )(logits_t, dx_t, dy_t, dw_t, dh_t, props_t)


def _nms_class(r, scores_hbm, x1_hbm, y1_hbm, x2_hbm, y2_hbm, keep_hbm,
               sv, x1v, y1v, x2v, y2v, kfull,
               idxc, scc, x1c, y1c, x2c, y2c, areac, keepc):
    pltpu.sync_copy(scores_hbm.at[r], sv)
    pltpu.sync_copy(x1_hbm.at[r], x1v)
    pltpu.sync_copy(y1_hbm.at[r], y1v)
    pltpu.sync_copy(x2_hbm.at[r], x2v)
    pltpu.sync_copy(y2_hbm.at[r], y2v)

    lane = lax.iota(jnp.int32, 16)

    def cblock(b, cnt):
        sl = pl.ds(b * 16, 16)
        s = sv[sl]
        kfull[sl] = jnp.zeros((16,), jnp.float32)
        m = s > SCORE_THRESH
        mi = m.astype(jnp.int32)
        cum = plsc.cumsum(mi)
        pos = jnp.full((16,), cnt, jnp.int32) + cum - mi
        plsc.store_scatter(idxc, [pos],
                           lane + jnp.full((16,), b * 16, jnp.int32), mask=m)
        return cnt + cum[15]

    cnt = lax.fori_loop(0, NP // 16, cblock, 0)

    idxc[pl.ds(cnt, 16)] = lane + N
    idxc[pl.ds(cnt + 16, 16)] = lane + N + 16
    nblk = (cnt + 15) // 16 + 1
    ntot = nblk * 16

    def gblock(b, carry):
        sl = pl.ds(b * 16, 16)
        ii = idxc[sl]
        x1 = plsc.load_gather(x1v, [ii])
        y1 = plsc.load_gather(y1v, [ii])
        x2 = plsc.load_gather(x2v, [ii])
        y2 = plsc.load_gather(y2v, [ii])
        x1c[sl] = x1
        y1c[sl] = y1
        x2c[sl] = x2
        y2c[sl] = y2
        areac[sl] = (x2 - x1) * (y2 - y1)
        scc[sl] = plsc.load_gather(sv, [ii])
        keepc[sl] = jnp.ones((16,), jnp.int32)
        return carry

    lax.fori_loop(0, nblk, gblock, 0)

    def sweep(_):
        def jblock(jb, acc):
            slj = pl.ds(jb * 16, 16)
            kv = keepc[slj]
            sjv = scc[slj]
            x1jv = x1c[slj]
            y1jv = y1c[slj]
            x2jv = x2c[slj]
            y2jv = y2c[slj]
            ajv = areac[slj]
            rjv = idxc[slj]
            spl = lambda v: jnp.full((16,), v[l], v.dtype)
            for l in range(16):
                sj = spl(sjv)
                x1j = spl(x1jv)
                y1j = spl(y1jv)
                x2j = spl(x2jv)
                y2j = spl(y2jv)
                aj = spl(ajv)
                rj = spl(rjv)

                def suppress(sj=sj, x1j=x1j, y1j=y1j, x2j=x2j, y2j=y2j,
                             aj=aj, rj=rj):
                    def bbody(b, a2):
                        sl = pl.ds(b * 16, 16)
                        si = scc[sl]
                        ri = idxc[sl]
                        prec = (sj > si) | ((sj == si) & (rj < ri))
                        xx1 = jnp.maximum(x1j, x1c[sl])
                        yy1 = jnp.maximum(y1j, y1c[sl])
                        xx2 = jnp.minimum(x2j, x2c[sl])
                        yy2 = jnp.minimum(y2j, y2c[sl])
                        inter = (jnp.maximum(xx2 - xx1, 0.0)
                                 * jnp.maximum(yy2 - yy1, 0.0))
                        iou = inter / jnp.maximum(aj + areac[sl] - inter, 1e-12)
                        sup = prec & (iou > NMS_THRESH)
                        ki = keepc[sl]
                        newk = jnp.where(sup, 0, ki)
                        keepc[sl] = newk
                        return a2 + jnp.sum(jnp.where(newk != ki, 1, 0))

                    return lax.fori_loop(0, nblk, bbody, 0)

                acc = acc + lax.cond(kv[l] > 0, suppress, lambda: 0)
            return acc

        return lax.fori_loop(0, nblk, jblock, 0)

    lax.while_loop(lambda ch: ch > 0, sweep, jnp.int32(1))

    def sblock(b, carry):
        sl = pl.ds(b * 16, 16)
        plsc.store_scatter(kfull, [idxc[sl]], keepc[sl].astype(jnp.float32))
        return carry

    lax.fori_loop(0, nblk, sblock, 0)
    pltpu.sync_copy(kfull, keep_hbm.at[r])


def _nms_sc_body(scores_hbm, x1_hbm, y1_hbm, x2_hbm, y2_hbm, keep_hbm,
                 sv, x1v, y1v, x2v, y2v, kfull,
                 idxc, scc, x1c, y1c, x2c, y2c, areac, keepc):
    wid = lax.axis_index("s") * 2 + lax.axis_index("c")
    for t in range(3):
        c = wid + 32 * t

        @pl.when(c < NCLS)
        def _():
            _nms_class(c, scores_hbm, x1_hbm, y1_hbm, x2_hbm, y2_hbm, keep_hbm,
                       sv, x1v, y1v, x2v, y2v, kfull,
                       idxc, scc, x1c, y1c, x2c, y2c, areac, keepc)


def _nms_sc(scores_t, x1_t, y1_t, x2_t, y2_t):
    mesh = plsc.VectorSubcoreMesh(core_axis_name="c", subcore_axis_name="s")
    f32 = jnp.float32
    kern = pl.kernel(
        _nms_sc_body,
        mesh=mesh,
        compiler_params=pltpu.CompilerParams(needs_layout_passes=False),
        out_type=jax.ShapeDtypeStruct((NCLS, NP), f32),
        scratch_types=[
            pltpu.VMEM((NP,), f32),
            pltpu.VMEM((NP,), f32),
            pltpu.VMEM((NP,), f32),
            pltpu.VMEM((NP,), f32),
            pltpu.VMEM((NP,), f32),
            pltpu.VMEM((NP,), f32),
            pltpu.VMEM((CAP,), jnp.int32),
            pltpu.VMEM((CAP,), f32),
            pltpu.VMEM((CAP,), f32),
            pltpu.VMEM((CAP,), f32),
            pltpu.VMEM((CAP,), f32),
            pltpu.VMEM((CAP,), f32),
            pltpu.VMEM((CAP,), f32),
            pltpu.VMEM((CAP,), jnp.int32),
        ],
    )
    return kern(scores_t, x1_t, y1_t, x2_t, y2_t)


def _merge_refresh(c, scores_ref, keep_ref, x1_ref, y1_ref, x2_ref, y2_ref,
                   best_s, best_r, bx1, by1, bx2, by2, last_s, last_r):
    row = pl.ds(c, 1)
    srow = scores_ref[row, :]
    krow = keep_ref[row, :]
    ls = last_s[row, :]
    lr = last_r[row, :]
    ri = lax.broadcasted_iota(jnp.int32, (1, NP), 1).astype(jnp.float32)
    cand = (krow > 0.0) & (srow > SCORE_THRESH) & (
        (srow < ls) | ((srow == ls) & (ri > lr)))
    ms = jnp.max(jnp.where(cand, srow, NEG))
    mr = jnp.min(jnp.where(cand & (srow == ms), ri, 1.0e9))
    pos = cand & (srow == ms) & (ri == mr)
    one = lambda v: jnp.broadcast_to(v, (1, 1))
    best_s[row, :] = one(ms)
    best_r[row, :] = one(mr)
    bx1[row, :] = one(jnp.sum(jnp.where(pos, x1_ref[row, :], 0.0)))
    by1[row, :] = one(jnp.sum(jnp.where(pos, y1_ref[row, :], 0.0)))
    bx2[row, :] = one(jnp.sum(jnp.where(pos, x2_ref[row, :], 0.0)))
    by2[row, :] = one(jnp.sum(jnp.where(pos, y2_ref[row, :], 0.0)))


def _merge_body(scores_ref, keep_ref, x1_ref, y1_ref, x2_ref, y2_ref,
                os_ref, ocls_ref, ox1_ref, oy1_ref, ox2_ref, oy2_ref,
                best_s, best_r, bx1, by1, bx2, by2, last_s, last_r):
    zeros = jnp.zeros((128, 1), jnp.float32)
    os_ref[...] = zeros
    ocls_ref[...] = zeros
    ox1_ref[...] = zeros
    oy1_ref[...] = zeros
    ox2_ref[...] = zeros
    oy2_ref[...] = zeros

    s_all = scores_ref[...]
    k_all = keep_ref[...]
    ri2 = lax.broadcasted_iota(jnp.int32, (NCLS, NP), 1).astype(jnp.float32)
    cand = (k_all > 0.0) & (s_all > SCORE_THRESH)
    ms = jnp.max(jnp.where(cand, s_all, NEG), axis=1, keepdims=True)
    mr = jnp.min(jnp.where(cand & (s_all == ms), ri2, 1.0e9),
                 axis=1, keepdims=True)
    pos = cand & (s_all == ms) & (ri2 == mr)
    best_s[...] = ms
    best_r[...] = mr
    bx1[...] = jnp.sum(jnp.where(pos, x1_ref[...], 0.0), axis=1, keepdims=True)
    by1[...] = jnp.sum(jnp.where(pos, y1_ref[...], 0.0), axis=1, keepdims=True)
    bx2[...] = jnp.sum(jnp.where(pos, x2_ref[...], 0.0), axis=1, keepdims=True)
    by2[...] = jnp.sum(jnp.where(pos, y2_ref[...], 0.0), axis=1, keepdims=True)
    last_s[...] = jnp.full((NCLS, 1), 3.0e38, jnp.float32)
    last_r[...] = jnp.full((NCLS, 1), -1.0, jnp.float32)

    cls_iota = lax.broadcasted_iota(jnp.int32, (NCLS, 1), 0).astype(jnp.float32)

    def kbody(k, carry):
        bs = best_s[...]
        mx = jnp.max(bs)
        ci_f = jnp.min(jnp.where(bs == mx, cls_iota, 1.0e9))
        valid = mx > 0.5 * NEG
        ci = jnp.where(valid, ci_f, 0.0).astype(jnp.int32)
        crow = pl.ds(ci, 1)
        mval = jnp.where(valid, 1.0, 0.0)

        def rd(ref):
            return ref[crow, :][0, 0]

        okslot = pl.ds(k, 1)
        one = lambda v: jnp.broadcast_to(v, (1, 1))
        os_ref[okslot, :] = one(jnp.where(valid, mx, 0.0))
        ocls_ref[okslot, :] = one(jnp.where(valid, ci_f + 1.0, 0.0))
        ox1_ref[okslot, :] = one(rd(bx1) * mval)
        oy1_ref[okslot, :] = one(rd(by1) * mval)
        ox2_ref[okslot, :] = one(rd(bx2) * mval)
        oy2_ref[okslot, :] = one(rd(by2) * mval)

        last_s[crow, :] = one(jnp.where(valid, mx, rd(last_s)))
        last_r[crow, :] = one(jnp.where(valid, rd(best_r), rd(last_r)))
        _merge_refresh(ci, scores_ref, keep_ref, x1_ref, y1_ref, x2_ref, y2_ref,
                       best_s, best_r, bx1, by1, bx2, by2, last_s, last_r)
        return carry

    lax.fori_loop(0, DETS, kbody, 0)


def _merge(scores_t, keep_t, x1_t, y1_t, x2_t, y2_t):
    f32 = jnp.float32
    out = jax.ShapeDtypeStruct((128, 1), f32)
    st = pltpu.VMEM((NCLS, 1), f32)
    return pl.pallas_call(
        _merge_body,
        out_shape=[out] * 6,
        scratch_shapes=[st] * 8,
    )(scores_t, keep_t, x1_t, y1_t, x2_t, y2_t)


def kernel(class_logits, box_regression, proposals):
    logits_t = jnp.pad(class_logits.T, ((0, 88 - NC), (0, NP - N)),
                       constant_values=NEG)
    regs = box_regression.reshape(N, NC, 4)[:, 1:, :]
    regs_t = jnp.transpose(regs, (2, 1, 0))
    pad = ((0, 0), (0, NP - N))
    dx_t = jnp.pad(regs_t[0], pad)
    dy_t = jnp.pad(regs_t[1], pad)
    dw_t = jnp.pad(regs_t[2], pad)
    dh_t = jnp.pad(regs_t[3], pad)
    props_t = jnp.pad(proposals.T, ((0, 4), (0, NP - N)))

    scores_t, x1_t, y1_t, x2_t, y2_t = _score_decode(
        logits_t, dx_t, dy_t, dw_t, dh_t, props_t)
    keep_t = _nms_sc(scores_t, x1_t, y1_t, x2_t, y2_t)
    os_, ocls, ox1, oy1, ox2, oy2 = _merge(
        scores_t, keep_t, x1_t, y1_t, x2_t, y2_t)

    dets = jnp.concatenate(
        [ox1[:DETS], oy1[:DETS], ox2[:DETS], oy2[:DETS], os_[:DETS]], axis=1)
    labels = ocls[:DETS, 0].astype(jnp.int32)
    return dets, labels


# speedup vs baseline: 928.3371x; 1.6409x over previous
import functools
import math

import jax
import jax.numpy as jnp
from jax import lax
from jax.experimental import pallas as pl
from jax.experimental.pallas import tpu as pltpu
from jax.experimental.pallas import tpu_sc as plsc

N = 5000
NP = 5120
NC = 81
NCLS = 80
SCORE_THRESH = 0.05
NMS_THRESH = 0.5
DETS = 100
CLIP = float(math.log(1000.0 / 16.0))
IMG_W, IMG_H = 1333.0, 800.0
NEG = -1.0e30
CAP = 5136


def _score_decode_body(logits_ref, dx_ref, dy_ref, dw_ref, dh_ref, props_ref,
                       sc_ref, x1_ref, y1_ref, x2_ref, y2_ref):
    l = logits_ref[...]
    m = jnp.max(l, axis=0, keepdims=True)
    e = jnp.exp(l - m)
    s = jnp.sum(e, axis=0, keepdims=True)
    prob = e / s
    col = lax.broadcasted_iota(jnp.int32, (1, NP), 1)
    sc_ref[...] = jnp.where(col < N, prob[1:NC], 0.0)

    px1 = props_ref[0:1, :]
    py1 = props_ref[1:2, :]
    px2 = props_ref[2:3, :]
    py2 = props_ref[3:4, :]
    w = px2 - px1 + 1.0
    h = py2 - py1 + 1.0
    cx = px1 + 0.5 * w
    cy = py1 + 0.5 * h
    dx = dx_ref[...] / 10.0
    dy = dy_ref[...] / 10.0
    dw = jnp.minimum(dw_ref[...] / 5.0, CLIP)
    dh = jnp.minimum(dh_ref[...] / 5.0, CLIP)
    pcx = dx * w + cx
    pcy = dy * h + cy
    pw = jnp.exp(dw) * w
    ph = jnp.exp(dh) * h
    x1_ref[...] = jnp.clip(pcx - 0.5 * pw, 0.0, IMG_W - 1.0)
    y1_ref[...] = jnp.clip(pcy - 0.5 * ph, 0.0, IMG_H - 1.0)
    x2_ref[...] = jnp.clip(pcx + 0.5 * pw - 1.0, 0.0, IMG_W - 1.0)
    y2_ref[...] = jnp.clip(pcy + 0.5 * ph - 1.0, 0.0, IMG_H - 1.0)


def _score_decode(logits_t, dx_t, dy_t, dw_t, dh_t, props_t):
    out = jax.ShapeDtypeStruct((NCLS, NP), jnp.float32)
    return pl.pallas_call(
        _score_decode_body,
        out_shape=[out] * 5,
    )(logits_t, dx_t, dy_t, dw_t, dh_t, props_t)


def _nms_class(r, scores_hbm, x1_hbm, y1_hbm, x2_hbm, y2_hbm, keep_hbm,
               sv, x1v, y1v, x2v, y2v, kfull,
               idxc, scc, x1c, y1c, x2c, y2c, areac, statec, unkc):
    pltpu.sync_copy(scores_hbm.at[r], sv)
    pltpu.sync_copy(x1_hbm.at[r], x1v)
    pltpu.sync_copy(y1_hbm.at[r], y1v)
    pltpu.sync_copy(x2_hbm.at[r], x2v)
    pltpu.sync_copy(y2_hbm.at[r], y2v)

    lane = lax.iota(jnp.int32, 16)

    def cblock(b, cnt):
        sl = pl.ds(b * 16, 16)
        s = sv[sl]
        kfull[sl] = jnp.zeros((16,), jnp.float32)
        m = s > SCORE_THRESH
        mi = m.astype(jnp.int32)
        cum = plsc.cumsum(mi)
        pos = jnp.full((16,), cnt, jnp.int32) + cum - mi
        plsc.store_scatter(idxc, [pos],
                           lane + jnp.full((16,), b * 16, jnp.int32), mask=m)
        return cnt + cum[15]

    cnt = lax.fori_loop(0, NP // 16, cblock, 0)

    idxc[pl.ds(cnt, 16)] = lane + N
    idxc[pl.ds(cnt + 16, 16)] = lane + N + 16
    nblk = (cnt + 15) // 16 + 1
    ntot = nblk * 16

    def gblock(b, carry):
        sl = pl.ds(b * 16, 16)
        ii = idxc[sl]
        x1 = plsc.load_gather(x1v, [ii])
        y1 = plsc.load_gather(y1v, [ii])
        x2 = plsc.load_gather(x2v, [ii])
        y2 = plsc.load_gather(y2v, [ii])
        x1c[sl] = x1
        y1c[sl] = y1
        x2c[sl] = x2
        y2c[sl] = y2
        areac[sl] = (x2 - x1) * (y2 - y1)
        scc[sl] = plsc.load_gather(sv, [ii])
        statec[sl] = jnp.zeros((16,), jnp.int32)
        return carry

    lax.fori_loop(0, nblk, gblock, 0)

    def r1block(jb, carry):
        slj = pl.ds(jb * 16, 16)
        sjv = scc[slj]
        x1jv = x1c[slj]
        y1jv = y1c[slj]
        x2jv = x2c[slj]
        y2jv = y2c[slj]
        ajv = areac[slj]
        rjv = idxc[slj]
        spl = lambda v: jnp.full((16,), v[l], v.dtype)
        for l in range(16):
            sj = spl(sjv)
            x1j = spl(x1jv)
            y1j = spl(y1jv)
            x2j = spl(x2jv)
            y2j = spl(y2jv)
            aj = spl(ajv)
            rj = spl(rjv)

            def bbody(b, c2, sj=sj, x1j=x1j, y1j=y1j, x2j=x2j, y2j=y2j,
                      aj=aj, rj=rj):
                sl = pl.ds(b * 16, 16)
                si = scc[sl]
                ri = idxc[sl]
                prec = (sj > si) | ((sj == si) & (rj < ri))
                xx1 = jnp.maximum(x1j, x1c[sl])
                yy1 = jnp.maximum(y1j, y1c[sl])
                xx2 = jnp.minimum(x2j, x2c[sl])
                yy2 = jnp.minimum(y2j, y2c[sl])
                inter = (jnp.maximum(xx2 - xx1, 0.0)
                         * jnp.maximum(yy2 - yy1, 0.0))
                iou = inter / jnp.maximum(aj + areac[sl] - inter, 1e-12)
                hit = prec & (iou > NMS_THRESH)
                statec[sl] = jnp.where(hit, 1, statec[sl])
                return c2

            lax.fori_loop(0, nblk, bbody, 0)
        return carry

    lax.fori_loop(0, nblk, r1block, 0)

    def cvblock(b, u):
        sl = pl.ds(b * 16, 16)
        contested = statec[sl] == 1
        statec[sl] = jnp.where(contested, 0, 1)
        ci = contested.astype(jnp.int32)
        cum = plsc.cumsum(ci)
        pos = jnp.full((16,), u, jnp.int32) + cum - ci
        plsc.store_scatter(unkc, [pos], lane + jnp.full((16,), b * 16, jnp.int32),
                           mask=contested)
        return u + cum[15]

    nunk = lax.fori_loop(0, nblk, cvblock, 0)
    unkc[pl.ds(nunk, 16)] = jnp.zeros((16,), jnp.int32)

    def rnd(u):
        nub = (u + 15) // 16

        def ublock(ub, newu):
            slu = pl.ds(ub * 16, 16)
            iv = unkc[slu]
            lanepos = lane + jnp.full((16,), ub * 16, jnp.int32)
            lvalid = lanepos < jnp.full((16,), u, jnp.int32)
            siv = plsc.load_gather(scc, [iv])
            x1iv = plsc.load_gather(x1c, [iv])
            y1iv = plsc.load_gather(y1c, [iv])
            x2iv = plsc.load_gather(x2c, [iv])
            y2iv = plsc.load_gather(y2c, [iv])
            aiv = plsc.load_gather(areac, [iv])
            riv = plsc.load_gather(idxc, [iv])
            newv = jnp.zeros((16,), jnp.int32)
            spl = lambda v: jnp.full((16,), v[l], v.dtype)
            for l in range(16):
                si = spl(siv)
                x1i = spl(x1iv)
                y1i = spl(y1iv)
                x2i = spl(x2iv)
                y2i = spl(y2iv)
                ai = spl(aiv)
                ri = spl(riv)

                def jb2(b, c2, si=si, x1i=x1i, y1i=y1i, x2i=x2i, y2i=y2i,
                        ai=ai, ri=ri):
                    sl = pl.ds(b * 16, 16)
                    sj = scc[sl]
                    rj = idxc[sl]
                    prec = (sj > si) | ((sj == si) & (rj < ri))
                    xx1 = jnp.maximum(x1i, x1c[sl])
                    yy1 = jnp.maximum(y1i, y1c[sl])
                    xx2 = jnp.minimum(x2i, x2c[sl])
                    yy2 = jnp.minimum(y2i, y2c[sl])
                    inter = (jnp.maximum(xx2 - xx1, 0.0)
                             * jnp.maximum(yy2 - yy1, 0.0))
                    iou = inter / jnp.maximum(ai + areac[sl] - inter, 1e-12)
                    ov = prec & (iou > NMS_THRESH)
                    st = statec[sl]
                    live = ov & (st != 2)
                    kept = ov & (st == 1)
                    return (c2[0] + jnp.sum(live.astype(jnp.int32)),
                            c2[1] + jnp.sum(kept.astype(jnp.int32)))

                nd, kp = lax.fori_loop(0, nblk, jb2, (0, 0))
                stl = jnp.where(kp > 0, 2, jnp.where(nd > 0, 0, 1))
                newv = jnp.where(lane == l, jnp.full((16,), stl, jnp.int32),
                                 newv)
            plsc.store_scatter(statec, [iv], newv, mask=lvalid)
            um = (newv == 0) & lvalid
            umi = um.astype(jnp.int32)
            cum = plsc.cumsum(umi)
            pos = jnp.full((16,), newu, jnp.int32) + cum - umi
            plsc.store_scatter(unkc, [pos], iv, mask=um)
            return newu + cum[15]

        return lax.fori_loop(0, nub, ublock, 0)

    lax.while_loop(lambda u: u > 0, rnd, nunk)

    def sblock(b, carry):
        sl = pl.ds(b * 16, 16)
        kf = (statec[sl] == 1).astype(jnp.float32)
        plsc.store_scatter(kfull, [idxc[sl]], kf)
        return carry

    lax.fori_loop(0, nblk, sblock, 0)
    pltpu.sync_copy(kfull, keep_hbm.at[r])


def _nms_sc_body(scores_hbm, x1_hbm, y1_hbm, x2_hbm, y2_hbm, keep_hbm,
                 sv, x1v, y1v, x2v, y2v, kfull,
                 idxc, scc, x1c, y1c, x2c, y2c, areac, statec, unkc):
    wid = lax.axis_index("s") * 2 + lax.axis_index("c")

    def per_class(t, carry):
        c = wid + 32 * t

        @pl.when(c < NCLS)
        def _():
            _nms_class(c, scores_hbm, x1_hbm, y1_hbm, x2_hbm, y2_hbm, keep_hbm,
                       sv, x1v, y1v, x2v, y2v, kfull,
                       idxc, scc, x1c, y1c, x2c, y2c, areac, statec, unkc)

        return carry

    lax.fori_loop(0, 3, per_class, 0)


def _nms_sc(scores_t, x1_t, y1_t, x2_t, y2_t):
    mesh = plsc.VectorSubcoreMesh(core_axis_name="c", subcore_axis_name="s")
    f32 = jnp.float32
    kern = pl.kernel(
        _nms_sc_body,
        mesh=mesh,
        compiler_params=pltpu.CompilerParams(needs_layout_passes=False),
        out_type=jax.ShapeDtypeStruct((NCLS, NP), f32),
        scratch_types=[
            pltpu.VMEM((NP,), f32),
            pltpu.VMEM((NP,), f32),
            pltpu.VMEM((NP,), f32),
            pltpu.VMEM((NP,), f32),
            pltpu.VMEM((NP,), f32),
            pltpu.VMEM((NP,), f32),
            pltpu.VMEM((CAP,), jnp.int32),
            pltpu.VMEM((CAP,), f32),
            pltpu.VMEM((CAP,), f32),
            pltpu.VMEM((CAP,), f32),
            pltpu.VMEM((CAP,), f32),
            pltpu.VMEM((CAP,), f32),
            pltpu.VMEM((CAP,), f32),
            pltpu.VMEM((CAP,), jnp.int32),
            pltpu.VMEM((CAP,), jnp.int32),
        ],
    )
    return kern(scores_t, x1_t, y1_t, x2_t, y2_t)


def _merge_refresh(c, scores_ref, keep_ref, x1_ref, y1_ref, x2_ref, y2_ref,
                   best_s, best_r, bx1, by1, bx2, by2, last_s, last_r):
    row = pl.ds(c, 1)
    srow = scores_ref[row, :]
    krow = keep_ref[row, :]
    ls = last_s[row, :]
    lr = last_r[row, :]
    ri = lax.broadcasted_iota(jnp.int32, (1, NP), 1).astype(jnp.float32)
    cand = (krow > 0.0) & (srow > SCORE_THRESH) & (
        (srow < ls) | ((srow == ls) & (ri > lr)))
    ms = jnp.max(jnp.where(cand, srow, NEG))
    mr = jnp.min(jnp.where(cand & (srow == ms), ri, 1.0e9))
    pos = cand & (srow == ms) & (ri == mr)
    one = lambda v: jnp.broadcast_to(v, (1, 1))
    best_s[row, :] = one(ms)
    best_r[row, :] = one(mr)
    bx1[row, :] = one(jnp.sum(jnp.where(pos, x1_ref[row, :], 0.0)))
    by1[row, :] = one(jnp.sum(jnp.where(pos, y1_ref[row, :], 0.0)))
    bx2[row, :] = one(jnp.sum(jnp.where(pos, x2_ref[row, :], 0.0)))
    by2[row, :] = one(jnp.sum(jnp.where(pos, y2_ref[row, :], 0.0)))


def _merge_body(scores_ref, keep_ref, x1_ref, y1_ref, x2_ref, y2_ref,
                os_ref, ocls_ref, ox1_ref, oy1_ref, ox2_ref, oy2_ref,
                best_s, best_r, bx1, by1, bx2, by2, last_s, last_r):
    zeros = jnp.zeros((128, 1), jnp.float32)
    os_ref[...] = zeros
    ocls_ref[...] = zeros
    ox1_ref[...] = zeros
    oy1_ref[...] = zeros
    ox2_ref[...] = zeros
    oy2_ref[...] = zeros

    s_all = scores_ref[...]
    k_all = keep_ref[...]
    ri2 = lax.broadcasted_iota(jnp.int32, (NCLS, NP), 1).astype(jnp.float32)
    cand = (k_all > 0.0) & (s_all > SCORE_THRESH)
    ms = jnp.max(jnp.where(cand, s_all, NEG), axis=1, keepdims=True)
    mr = jnp.min(jnp.where(cand & (s_all == ms), ri2, 1.0e9),
                 axis=1, keepdims=True)
    pos = cand & (s_all == ms) & (ri2 == mr)
    best_s[...] = ms
    best_r[...] = mr
    bx1[...] = jnp.sum(jnp.where(pos, x1_ref[...], 0.0), axis=1, keepdims=True)
    by1[...] = jnp.sum(jnp.where(pos, y1_ref[...], 0.0), axis=1, keepdims=True)
    bx2[...] = jnp.sum(jnp.where(pos, x2_ref[...], 0.0), axis=1, keepdims=True)
    by2[...] = jnp.sum(jnp.where(pos, y2_ref[...], 0.0), axis=1, keepdims=True)
    last_s[...] = jnp.full((NCLS, 1), 3.0e38, jnp.float32)
    last_r[...] = jnp.full((NCLS, 1), -1.0, jnp.float32)

    cls_iota = lax.broadcasted_iota(jnp.int32, (NCLS, 1), 0).astype(jnp.float32)

    def kbody(k, carry):
        bs = best_s[...]
        mx = jnp.max(bs)
        ci_f = jnp.min(jnp.where(bs == mx, cls_iota, 1.0e9))
        valid = mx > 0.5 * NEG
        ci = jnp.where(valid, ci_f, 0.0).astype(jnp.int32)
        crow = pl.ds(ci, 1)
        mval = jnp.where(valid, 1.0, 0.0)

        def rd(ref):
            return ref[crow, :][0, 0]

        okslot = pl.ds(k, 1)
        one = lambda v: jnp.broadcast_to(v, (1, 1))
        os_ref[okslot, :] = one(jnp.where(valid, mx, 0.0))
        ocls_ref[okslot, :] = one(jnp.where(valid, ci_f + 1.0, 0.0))
        ox1_ref[okslot, :] = one(rd(bx1) * mval)
        oy1_ref[okslot, :] = one(rd(by1) * mval)
        ox2_ref[okslot, :] = one(rd(bx2) * mval)
        oy2_ref[okslot, :] = one(rd(by2) * mval)

        last_s[crow, :] = one(jnp.where(valid, mx, rd(last_s)))
        last_r[crow, :] = one(jnp.where(valid, rd(best_r), rd(last_r)))
        _merge_refresh(ci, scores_ref, keep_ref, x1_ref, y1_ref, x2_ref, y2_ref,
                       best_s, best_r, bx1, by1, bx2, by2, last_s, last_r)
        return carry

    lax.fori_loop(0, DETS, kbody, 0)


def _merge(scores_t, keep_t, x1_t, y1_t, x2_t, y2_t):
    f32 = jnp.float32
    out = jax.ShapeDtypeStruct((128, 1), f32)
    st = pltpu.VMEM((NCLS, 1), f32)
    return pl.pallas_call(
        _merge_body,
        out_shape=[out] * 6,
        scratch_shapes=[st] * 8,
    )(scores_t, keep_t, x1_t, y1_t, x2_t, y2_t)


def kernel(class_logits, box_regression, proposals):
    logits_t = jnp.pad(class_logits.T, ((0, 88 - NC), (0, NP - N)),
                       constant_values=NEG)
    regs = box_regression.reshape(N, NC, 4)[:, 1:, :]
    regs_t = jnp.transpose(regs, (2, 1, 0))
    pad = ((0, 0), (0, NP - N))
    dx_t = jnp.pad(regs_t[0], pad)
    dy_t = jnp.pad(regs_t[1], pad)
    dw_t = jnp.pad(regs_t[2], pad)
    dh_t = jnp.pad(regs_t[3], pad)
    props_t = jnp.pad(proposals.T, ((0, 4), (0, NP - N)))

    scores_t, x1_t, y1_t, x2_t, y2_t = _score_decode(
        logits_t, dx_t, dy_t, dw_t, dh_t, props_t)
    keep_t = _nms_sc(scores_t, x1_t, y1_t, x2_t, y2_t)
    os_, ocls, ox1, oy1, ox2, oy2 = _merge(
        scores_t, keep_t, x1_t, y1_t, x2_t, y2_t)

    dets = jnp.concatenate(
        [ox1[:DETS], oy1[:DETS], ox2[:DETS], oy2[:DETS], os_[:DETS]], axis=1)
    labels = ocls[:DETS, 0].astype(jnp.int32)
    return dets, labels


# speedup vs baseline: 975.4754x; 1.0508x over previous
import functools
import math

import jax
import jax.numpy as jnp
from jax import lax
from jax.experimental import pallas as pl
from jax.experimental.pallas import tpu as pltpu
from jax.experimental.pallas import tpu_sc as plsc

N = 5000
NP = 5120
NC = 81
NCLS = 80
SCORE_THRESH = 0.05
NMS_THRESH = 0.5
DETS = 100
CLIP = float(math.log(1000.0 / 16.0))
IMG_W, IMG_H = 1333.0, 800.0
NEG = -1.0e30
CAP = 5136


def _score_decode_body(logits_ref, dx_ref, dy_ref, dw_ref, dh_ref, props_ref,
                       sc_ref, x1_ref, y1_ref, x2_ref, y2_ref):
    l = logits_ref[...]
    m = jnp.max(l, axis=0, keepdims=True)
    e = jnp.exp(l - m)
    s = jnp.sum(e, axis=0, keepdims=True)
    prob = e / s
    col = lax.broadcasted_iota(jnp.int32, (1, NP), 1)
    sc_ref[...] = jnp.where(col < N, prob[1:NC], 0.0)

    px1 = props_ref[0:1, :]
    py1 = props_ref[1:2, :]
    px2 = props_ref[2:3, :]
    py2 = props_ref[3:4, :]
    w = px2 - px1 + 1.0
    h = py2 - py1 + 1.0
    cx = px1 + 0.5 * w
    cy = py1 + 0.5 * h
    dx = dx_ref[...] / 10.0
    dy = dy_ref[...] / 10.0
    dw = jnp.minimum(dw_ref[...] / 5.0, CLIP)
    dh = jnp.minimum(dh_ref[...] / 5.0, CLIP)
    pcx = dx * w + cx
    pcy = dy * h + cy
    pw = jnp.exp(dw) * w
    ph = jnp.exp(dh) * h
    x1_ref[...] = jnp.clip(pcx - 0.5 * pw, 0.0, IMG_W - 1.0)
    y1_ref[...] = jnp.clip(pcy - 0.5 * ph, 0.0, IMG_H - 1.0)
    x2_ref[...] = jnp.clip(pcx + 0.5 * pw - 1.0, 0.0, IMG_W - 1.0)
    y2_ref[...] = jnp.clip(pcy + 0.5 * ph - 1.0, 0.0, IMG_H - 1.0)


def _score_decode(logits_t, dx_t, dy_t, dw_t, dh_t, props_t):
    out = jax.ShapeDtypeStruct((NCLS, NP), jnp.float32)
    return pl.pallas_call(
        _score_decode_body,
        out_shape=[out] * 5,
    )(logits_t, dx_t, dy_t, dw_t, dh_t, props_t)


def _nms_class(r, scores_hbm, x1_hbm, y1_hbm, x2_hbm, y2_hbm, keep_hbm,
               sv, x1v, y1v, x2v, y2v, kfull,
               idxc, scc, x1c, y1c, x2c, y2c, areac, statec, unkc):
    pltpu.sync_copy(scores_hbm.at[r], sv)
    pltpu.sync_copy(x1_hbm.at[r], x1v)
    pltpu.sync_copy(y1_hbm.at[r], y1v)
    pltpu.sync_copy(x2_hbm.at[r], x2v)
    pltpu.sync_copy(y2_hbm.at[r], y2v)

    lane = lax.iota(jnp.int32, 16)

    def cblock(b, cnt):
        sl = pl.ds(b * 16, 16)
        s = sv[sl]
        kfull[sl] = jnp.zeros((16,), jnp.float32)
        m = s > SCORE_THRESH
        mi = m.astype(jnp.int32)
        cum = plsc.cumsum(mi)
        pos = jnp.full((16,), cnt, jnp.int32) + cum - mi
        plsc.store_scatter(idxc, [pos],
                           lane + jnp.full((16,), b * 16, jnp.int32), mask=m)
        return cnt + cum[15]

    cnt = lax.fori_loop(0, NP // 16, cblock, 0)

    idxc[pl.ds(cnt, 16)] = lane + N
    idxc[pl.ds(cnt + 16, 16)] = lane + N + 16
    nblk = (cnt + 15) // 16 + 1
    ntot = nblk * 16

    def gblock(b, carry):
        sl = pl.ds(b * 16, 16)
        ii = idxc[sl]
        x1 = plsc.load_gather(x1v, [ii])
        y1 = plsc.load_gather(y1v, [ii])
        x2 = plsc.load_gather(x2v, [ii])
        y2 = plsc.load_gather(y2v, [ii])
        x1c[sl] = x1
        y1c[sl] = y1
        x2c[sl] = x2
        y2c[sl] = y2
        areac[sl] = (x2 - x1) * (y2 - y1)
        scc[sl] = plsc.load_gather(sv, [ii])
        statec[sl] = jnp.zeros((16,), jnp.int32)
        return carry

    lax.fori_loop(0, nblk, gblock, 0)

    def r1block(jb, carry):
        slj = pl.ds(jb * 16, 16)
        sjv = scc[slj]
        x1jv = x1c[slj]
        y1jv = y1c[slj]
        x2jv = x2c[slj]
        y2jv = y2c[slj]
        ajv = areac[slj]
        rjv = idxc[slj]
        spl = lambda v: jnp.full((16,), v[l], v.dtype)
        for l in range(16):
            sj = spl(sjv)
            x1j = spl(x1jv)
            y1j = spl(y1jv)
            x2j = spl(x2jv)
            y2j = spl(y2jv)
            aj = spl(ajv)
            rj = spl(rjv)

            def bbody(b, c2, sj=sj, x1j=x1j, y1j=y1j, x2j=x2j, y2j=y2j,
                      aj=aj, rj=rj):
                sl = pl.ds(b * 16, 16)
                si = scc[sl]
                ri = idxc[sl]
                prec = (sj > si) | ((sj == si) & (rj < ri))
                xx1 = jnp.maximum(x1j, x1c[sl])
                yy1 = jnp.maximum(y1j, y1c[sl])
                xx2 = jnp.minimum(x2j, x2c[sl])
                yy2 = jnp.minimum(y2j, y2c[sl])
                inter = (jnp.maximum(xx2 - xx1, 0.0)
                         * jnp.maximum(yy2 - yy1, 0.0))
                iou = inter / jnp.maximum(aj + areac[sl] - inter, 1e-12)
                hit = prec & (iou > NMS_THRESH)
                statec[sl] = jnp.where(hit, 1, statec[sl])
                return c2

            lax.fori_loop(0, nblk, bbody, 0)
        return carry

    lax.fori_loop(0, nblk, r1block, 0)

    def cvblock(b, u):
        sl = pl.ds(b * 16, 16)
        contested = statec[sl] == 1
        statec[sl] = jnp.where(contested, 0, 1)
        ci = contested.astype(jnp.int32)
        cum = plsc.cumsum(ci)
        pos = jnp.full((16,), u, jnp.int32) + cum - ci
        plsc.store_scatter(unkc, [pos], lane + jnp.full((16,), b * 16, jnp.int32),
                           mask=contested)
        return u + cum[15]

    nunk = lax.fori_loop(0, nblk, cvblock, 0)
    unkc[pl.ds(nunk, 16)] = jnp.zeros((16,), jnp.int32)

    def rnd(u):
        nub = (u + 15) // 16

        def ublock(ub, newu):
            slu = pl.ds(ub * 16, 16)
            iv = unkc[slu]
            lanepos = lane + jnp.full((16,), ub * 16, jnp.int32)
            lvalid = lanepos < jnp.full((16,), u, jnp.int32)
            siv = plsc.load_gather(scc, [iv])
            x1iv = plsc.load_gather(x1c, [iv])
            y1iv = plsc.load_gather(y1c, [iv])
            x2iv = plsc.load_gather(x2c, [iv])
            y2iv = plsc.load_gather(y2c, [iv])
            aiv = plsc.load_gather(areac, [iv])
            riv = plsc.load_gather(idxc, [iv])
            newv = jnp.zeros((16,), jnp.int32)
            spl = lambda v: jnp.full((16,), v[l], v.dtype)
            for l in range(16):
                si = spl(siv)
                x1i = spl(x1iv)
                y1i = spl(y1iv)
                x2i = spl(x2iv)
                y2i = spl(y2iv)
                ai = spl(aiv)
                ri = spl(riv)

                def jb2(b, c2, si=si, x1i=x1i, y1i=y1i, x2i=x2i, y2i=y2i,
                        ai=ai, ri=ri):
                    sl = pl.ds(b * 16, 16)
                    sj = scc[sl]
                    rj = idxc[sl]
                    prec = (sj > si) | ((sj == si) & (rj < ri))
                    xx1 = jnp.maximum(x1i, x1c[sl])
                    yy1 = jnp.maximum(y1i, y1c[sl])
                    xx2 = jnp.minimum(x2i, x2c[sl])
                    yy2 = jnp.minimum(y2i, y2c[sl])
                    inter = (jnp.maximum(xx2 - xx1, 0.0)
                             * jnp.maximum(yy2 - yy1, 0.0))
                    iou = inter / jnp.maximum(ai + areac[sl] - inter, 1e-12)
                    ov = prec & (iou > NMS_THRESH)
                    st = statec[sl]
                    live = ov & (st != 2)
                    kept = ov & (st == 1)
                    return (c2[0] + jnp.sum(live.astype(jnp.int32)),
                            c2[1] + jnp.sum(kept.astype(jnp.int32)))

                nd, kp = lax.fori_loop(0, nblk, jb2, (0, 0))
                stl = jnp.where(kp > 0, 2, jnp.where(nd > 0, 0, 1))
                newv = jnp.where(lane == l, jnp.full((16,), stl, jnp.int32),
                                 newv)
            plsc.store_scatter(statec, [iv], newv, mask=lvalid)
            um = (newv == 0) & lvalid
            umi = um.astype(jnp.int32)
            cum = plsc.cumsum(umi)
            pos = jnp.full((16,), newu, jnp.int32) + cum - umi
            plsc.store_scatter(unkc, [pos], iv, mask=um)
            return newu + cum[15]

        return lax.fori_loop(0, nub, ublock, 0)

    lax.while_loop(lambda u: u > 0, rnd, nunk)

    def sblock(b, carry):
        sl = pl.ds(b * 16, 16)
        kf = (statec[sl] == 1).astype(jnp.float32)
        plsc.store_scatter(kfull, [idxc[sl]], kf)
        return carry

    lax.fori_loop(0, nblk, sblock, 0)
    pltpu.sync_copy(kfull, keep_hbm.at[r])


def _nms_sc_body(scores_hbm, x1_hbm, y1_hbm, x2_hbm, y2_hbm, keep_hbm,
                 sv, x1v, y1v, x2v, y2v, kfull,
                 idxc, scc, x1c, y1c, x2c, y2c, areac, statec, unkc):
    wid = lax.axis_index("s") * 2 + lax.axis_index("c")

    def per_class(t, carry):
        c = wid + 32 * t

        @pl.when(c < NCLS)
        def _():
            _nms_class(c, scores_hbm, x1_hbm, y1_hbm, x2_hbm, y2_hbm, keep_hbm,
                       sv, x1v, y1v, x2v, y2v, kfull,
                       idxc, scc, x1c, y1c, x2c, y2c, areac, statec, unkc)

        return carry

    lax.fori_loop(0, 3, per_class, 0)


def _nms_sc(scores_t, x1_t, y1_t, x2_t, y2_t):
    mesh = plsc.VectorSubcoreMesh(core_axis_name="c", subcore_axis_name="s")
    f32 = jnp.float32
    kern = pl.kernel(
        _nms_sc_body,
        mesh=mesh,
        compiler_params=pltpu.CompilerParams(needs_layout_passes=False),
        out_type=jax.ShapeDtypeStruct((NCLS, NP), f32),
        scratch_types=[
            pltpu.VMEM((NP,), f32),
            pltpu.VMEM((NP,), f32),
            pltpu.VMEM((NP,), f32),
            pltpu.VMEM((NP,), f32),
            pltpu.VMEM((NP,), f32),
            pltpu.VMEM((NP,), f32),
            pltpu.VMEM((CAP,), jnp.int32),
            pltpu.VMEM((CAP,), f32),
            pltpu.VMEM((CAP,), f32),
            pltpu.VMEM((CAP,), f32),
            pltpu.VMEM((CAP,), f32),
            pltpu.VMEM((CAP,), f32),
            pltpu.VMEM((CAP,), f32),
            pltpu.VMEM((CAP,), jnp.int32),
            pltpu.VMEM((CAP,), jnp.int32),
        ],
    )
    return kern(scores_t, x1_t, y1_t, x2_t, y2_t)


_SL, _LN = 8, NP // 8


def _page_iota():
    s = lax.broadcasted_iota(jnp.int32, (1, _SL, _LN), 1)
    l = lax.broadcasted_iota(jnp.int32, (1, _SL, _LN), 2)
    return (s * _LN + l).astype(jnp.float32)


def _merge_refresh(c, scores_ref, keep_ref, x1_ref, y1_ref, x2_ref, y2_ref,
                   best_s, best_r, bx1, by1, bx2, by2, last_s, last_r):
    row = pl.ds(c, 1)
    srow = scores_ref[row, :, :]
    krow = keep_ref[row, :, :]
    ls = last_s[row, :][0, 0]
    lr = last_r[row, :][0, 0]
    ri = _page_iota()
    cand = (krow > 0.0) & (srow > SCORE_THRESH) & (
        (srow < ls) | ((srow == ls) & (ri > lr)))
    ms = jnp.max(jnp.where(cand, srow, NEG))
    mr = jnp.min(jnp.where(cand & (srow == ms), ri, 1.0e9))
    pos = cand & (srow == ms) & (ri == mr)
    one = lambda v: jnp.broadcast_to(v, (1, 1))
    best_s[row, :] = one(ms)
    best_r[row, :] = one(mr)
    bx1[row, :] = one(jnp.sum(jnp.where(pos, x1_ref[row, :, :], 0.0)))
    by1[row, :] = one(jnp.sum(jnp.where(pos, y1_ref[row, :, :], 0.0)))
    bx2[row, :] = one(jnp.sum(jnp.where(pos, x2_ref[row, :, :], 0.0)))
    by2[row, :] = one(jnp.sum(jnp.where(pos, y2_ref[row, :, :], 0.0)))


def _merge_body(scores_ref, keep_ref, x1_ref, y1_ref, x2_ref, y2_ref,
                os_ref, ocls_ref, ox1_ref, oy1_ref, ox2_ref, oy2_ref,
                best_s, best_r, bx1, by1, bx2, by2, last_s, last_r):
    zeros = jnp.zeros((128, 1), jnp.float32)
    os_ref[...] = zeros
    ocls_ref[...] = zeros
    ox1_ref[...] = zeros
    oy1_ref[...] = zeros
    ox2_ref[...] = zeros
    oy2_ref[...] = zeros

    s_all = scores_ref[...]
    k_all = keep_ref[...]
    si2 = lax.broadcasted_iota(jnp.int32, (NCLS, _SL, _LN), 1)
    li2 = lax.broadcasted_iota(jnp.int32, (NCLS, _SL, _LN), 2)
    ri2 = (si2 * _LN + li2).astype(jnp.float32)
    cand = (k_all > 0.0) & (s_all > SCORE_THRESH)
    ms3 = jnp.max(jnp.max(jnp.where(cand, s_all, NEG), axis=2, keepdims=True),
                  axis=1, keepdims=True)
    mr3 = jnp.min(jnp.min(jnp.where(cand & (s_all == ms3), ri2, 1.0e9),
                          axis=2, keepdims=True), axis=1, keepdims=True)
    pos = cand & (s_all == ms3) & (ri2 == mr3)

    def red2(x):
        return jnp.sum(jnp.sum(x, axis=2, keepdims=True), axis=1)

    best_s[...] = jnp.reshape(ms3, (NCLS, 1))
    best_r[...] = jnp.reshape(mr3, (NCLS, 1))
    bx1[...] = red2(jnp.where(pos, x1_ref[...], 0.0))
    by1[...] = red2(jnp.where(pos, y1_ref[...], 0.0))
    bx2[...] = red2(jnp.where(pos, x2_ref[...], 0.0))
    by2[...] = red2(jnp.where(pos, y2_ref[...], 0.0))
    last_s[...] = jnp.full((NCLS, 1), 3.0e38, jnp.float32)
    last_r[...] = jnp.full((NCLS, 1), -1.0, jnp.float32)

    cls_iota = lax.broadcasted_iota(jnp.int32, (NCLS, 1), 0).astype(jnp.float32)

    def kbody(k, carry):
        bs = best_s[...]
        mx = jnp.max(bs)
        ci_f = jnp.min(jnp.where(bs == mx, cls_iota, 1.0e9))
        valid = mx > 0.5 * NEG
        ci = jnp.where(valid, ci_f, 0.0).astype(jnp.int32)
        crow = pl.ds(ci, 1)
        mval = jnp.where(valid, 1.0, 0.0)

        def rd(ref):
            return ref[crow, :][0, 0]

        okslot = pl.ds(k, 1)
        one = lambda v: jnp.broadcast_to(v, (1, 1))
        os_ref[okslot, :] = one(jnp.where(valid, mx, 0.0))
        ocls_ref[okslot, :] = one(jnp.where(valid, ci_f + 1.0, 0.0))
        ox1_ref[okslot, :] = one(rd(bx1) * mval)
        oy1_ref[okslot, :] = one(rd(by1) * mval)
        ox2_ref[okslot, :] = one(rd(bx2) * mval)
        oy2_ref[okslot, :] = one(rd(by2) * mval)

        last_s[crow, :] = one(jnp.where(valid, mx, rd(last_s)))
        last_r[crow, :] = one(jnp.where(valid, rd(best_r), rd(last_r)))
        _merge_refresh(ci, scores_ref, keep_ref, x1_ref, y1_ref, x2_ref, y2_ref,
                       best_s, best_r, bx1, by1, bx2, by2, last_s, last_r)
        return carry

    lax.fori_loop(0, DETS, kbody, 0)


def _merge(scores_t, keep_t, x1_t, y1_t, x2_t, y2_t):
    f32 = jnp.float32
    out = jax.ShapeDtypeStruct((128, 1), f32)
    st = pltpu.VMEM((NCLS, 1), f32)
    return pl.pallas_call(
        _merge_body,
        out_shape=[out] * 6,
        scratch_shapes=[st] * 8,
    )(scores_t, keep_t, x1_t, y1_t, x2_t, y2_t)


def kernel(class_logits, box_regression, proposals):
    logits_t = jnp.pad(class_logits.T, ((0, 88 - NC), (0, NP - N)),
                       constant_values=NEG)
    regs = box_regression.reshape(N, NC, 4)[:, 1:, :]
    regs_t = jnp.transpose(regs, (2, 1, 0))
    pad = ((0, 0), (0, NP - N))
    dx_t = jnp.pad(regs_t[0], pad)
    dy_t = jnp.pad(regs_t[1], pad)
    dw_t = jnp.pad(regs_t[2], pad)
    dh_t = jnp.pad(regs_t[3], pad)
    props_t = jnp.pad(proposals.T, ((0, 4), (0, NP - N)))

    scores_t, x1_t, y1_t, x2_t, y2_t = _score_decode(
        logits_t, dx_t, dy_t, dw_t, dh_t, props_t)
    keep_t = _nms_sc(scores_t, x1_t, y1_t, x2_t, y2_t)
    pg = lambda a: a.reshape(NCLS, _SL, _LN)
    os_, ocls, ox1, oy1, ox2, oy2 = _merge(
        pg(scores_t), pg(keep_t), pg(x1_t), pg(y1_t), pg(x2_t), pg(y2_t))

    dets = jnp.concatenate(
        [ox1[:DETS], oy1[:DETS], ox2[:DETS], oy2[:DETS], os_[:DETS]], axis=1)
    labels = ocls[:DETS, 0].astype(jnp.int32)
    return dets, labels
